# Initial kernel scaffold; baseline (speedup 1.0000x reference)
import jax
import jax.numpy as jnp
from jax.experimental import pallas as pl


def kernel(hits, cond, group_ids):
    raise NotImplementedError("write your pallas kernel here")


# speedup vs baseline: 1.8830x; 1.8830x over previous
import functools

import jax
import jax.numpy as jnp
from jax import lax
from jax.experimental import pallas as pl
from jax.experimental.pallas import tpu as pltpu
from jax.experimental.pallas import tpu_sc as plsc

L = 16
NC = 2
NS = 16
NW = NC * NS

TOTAL = 32768
F = 128
B_PER_W = TOTAL // NW
CHUNK = 128
N_CHUNKS = B_PER_W // CHUNK


def _sc_body(hits_hbm, cond_hbm, gid_hbm, out_hbm, idx_v, hits_v, cond_v, sem):
    wid = lax.axis_index("s") * NC + lax.axis_index("c")
    base = wid * B_PER_W

    pltpu.sync_copy(gid_hbm.at[pl.ds(wid * N_CHUNKS, N_CHUNKS)], idx_v)

    for j in range(N_CHUNKS):
        row0 = base + j * CHUNK
        gather = pltpu.async_copy(cond_hbm.at[idx_v.at[j]], cond_v, sem)
        pltpu.sync_copy(hits_hbm.at[pl.ds(row0, CHUNK)], hits_v)
        gather.wait()

        def sub_row(r, carry):
            for g in range(F // L):
                s = pl.ds(g * L, L)
                cond_v[r, s] = hits_v[r, s] - cond_v[r, s]
            return carry

        lax.fori_loop(0, CHUNK, sub_row, 0)
        pltpu.sync_copy(cond_v, out_hbm.at[pl.ds(row0, CHUNK)])


@jax.jit
def _call(hits, cond, gid2d):
    mesh = plsc.VectorSubcoreMesh(core_axis_name="c", subcore_axis_name="s")
    k = pl.kernel(
        _sc_body,
        mesh=mesh,
        out_type=jax.ShapeDtypeStruct((TOTAL, F), jnp.float32),
        scratch_types=[
            pltpu.VMEM((N_CHUNKS, CHUNK), jnp.int32),
            pltpu.VMEM((CHUNK, F), jnp.float32),
            pltpu.VMEM((CHUNK, F), jnp.float32),
            pltpu.SemaphoreType.DMA,
        ],
    )
    return k(hits, cond, gid2d)


def kernel(hits, cond, group_ids):
    gid2d = group_ids.astype(jnp.int32).reshape(TOTAL // CHUNK, CHUNK)
    return _call(hits, cond, gid2d)


# speedup vs baseline: 2.3715x; 1.2594x over previous
import functools

import jax
import jax.numpy as jnp
from jax import lax
from jax.experimental import pallas as pl
from jax.experimental.pallas import tpu as pltpu
from jax.experimental.pallas import tpu_sc as plsc

L = 16
NC = 2
NS = 16
NW = NC * NS

TOTAL = 32768
F = 128
B_PER_W = TOTAL // NW
CHUNK = 128
N_CHUNKS = B_PER_W // CHUNK


def _sc_body(hits_hbm, cond_hbm, gid_hbm, out_hbm,
             idx_v, hits_v, cond_v, out_v, hsem, gsem, osem):
    wid = lax.axis_index("s") * NC + lax.axis_index("c")
    base = wid * B_PER_W

    pltpu.sync_copy(gid_hbm.at[pl.ds(wid * N_CHUNKS, N_CHUNKS)], idx_v)

    def start_inputs(j):
        b = j & 1
        row0 = base + j * CHUNK
        pltpu.async_copy(cond_hbm.at[idx_v.at[j]], cond_v.at[b], gsem)
        pltpu.async_copy(hits_hbm.at[pl.ds(row0, CHUNK)], hits_v.at[b], hsem)

    start_inputs(0)
    start_inputs(1)

    stores = []
    for j in range(N_CHUNKS):
        b = j & 1
        row0 = base + j * CHUNK
        pltpu.make_async_copy(cond_hbm.at[idx_v.at[j]], cond_v.at[b], gsem).wait()
        pltpu.make_async_copy(hits_hbm.at[pl.ds(row0, CHUNK)], hits_v.at[b],
                              hsem).wait()
        if j >= 2:
            stores[j - 2].wait()

        def sub_row(r, carry):
            for g in range(F // L):
                s = pl.ds(g * L, L)
                out_v[b, r, s] = hits_v[b, r, s] - cond_v[b, r, s]
            return carry

        lax.fori_loop(0, CHUNK, sub_row, 0)
        stores.append(
            pltpu.async_copy(out_v.at[b], out_hbm.at[pl.ds(row0, CHUNK)], osem))
        if j + 2 < N_CHUNKS:
            start_inputs(j + 2)

    stores[N_CHUNKS - 2].wait()
    stores[N_CHUNKS - 1].wait()


@jax.jit
def _call(hits, cond, gid2d):
    mesh = plsc.VectorSubcoreMesh(core_axis_name="c", subcore_axis_name="s")
    k = pl.kernel(
        _sc_body,
        mesh=mesh,
        out_type=jax.ShapeDtypeStruct((TOTAL, F), jnp.float32),
        scratch_types=[
            pltpu.VMEM((N_CHUNKS, CHUNK), jnp.int32),
            pltpu.VMEM((2, CHUNK, F), jnp.float32),
            pltpu.VMEM((2, CHUNK, F), jnp.float32),
            pltpu.VMEM((2, CHUNK, F), jnp.float32),
            pltpu.SemaphoreType.DMA,
            pltpu.SemaphoreType.DMA,
            pltpu.SemaphoreType.DMA,
        ],
    )
    return k(hits, cond, gid2d)


def kernel(hits, cond, group_ids):
    gid2d = group_ids.astype(jnp.int32).reshape(TOTAL // CHUNK, CHUNK)
    return _call(hits, cond, gid2d)
